# baseline (speedup 1.0000x reference)
import functools

import jax
import jax.numpy as jnp
from jax import lax
from jax.experimental import pallas as pl
from jax.experimental.pallas import tpu as pltpu
from jax.experimental.pallas import tpu_sc as plsc

D = 49152
N = 1024
BASE = 128
HALF = BASE // 2
BD = 768
SPLITS = (6144, 6144, 12288, 12288, 12288)

_INFO = plsc.get_sparse_core_info()
_NC, _NS, _L = _INFO.num_cores, _INFO.num_subcores, _INFO.num_lanes
_NW = _NC * _NS
_HIMASK = -65536


_sc_mesh = plsc.VectorSubcoreMesh(core_axis_name="c", subcore_axis_name="s")


def _make_sc_gather(dsplit):
    rpw = dsplit // _NW
    chunk = 128 if rpw % 128 == 0 else 64
    nchunk = rpw // chunk
    nbuf = min(nchunk, 3)
    nobuf = min(nchunk, 2)

    @functools.partial(
        pl.kernel,
        mesh=_sc_mesh,
        out_type=jax.ShapeDtypeStruct((dsplit, BASE), jnp.float32),
        scratch_types=[
            pltpu.VMEM((rpw,), jnp.int32),
            pltpu.VMEM((rpw,), jnp.int32),
            pltpu.VMEM((nbuf, chunk, HALF), jnp.int32),
            pltpu.VMEM((nbuf, chunk, HALF), jnp.int32),
            pltpu.VMEM((nobuf, chunk, BASE), jnp.float32),
            pltpu.SemaphoreType.DMA((nbuf,)),
            pltpu.SemaphoreType.DMA((nobuf,)),
        ],
        compiler_params=pltpu.CompilerParams(use_tc_tiling_on_sc=False,
                                             needs_layout_passes=False),
    )
    def _sc_gather(w1_hbm, w2_hbm, f1_hbm, f2_hbm, g_hbm,
                   idx1_v, idx2_v, b1, b2, ob, gsem, wsem):
        wid = lax.axis_index("s") * _NC + lax.axis_index("c")
        base = wid * rpw
        pltpu.sync_copy(f1_hbm.at[pl.ds(base, rpw)], idx1_v)
        pltpu.sync_copy(f2_hbm.at[pl.ds(base, rpw)], idx2_v)

        gath = []
        for j in range(nchunk):
            s = j % nbuf
            off = j * chunk
            gath.append((
                pltpu.async_copy(
                    w1_hbm.at[idx1_v.at[pl.ds(off, chunk)]], b1.at[s],
                    gsem.at[s]),
                pltpu.async_copy(
                    w2_hbm.at[idx2_v.at[pl.ds(off, chunk)]], b2.at[s],
                    gsem.at[s]),
            ))

        wrt = [None] * nchunk
        for j in range(nchunk):
            s = j % nbuf
            o = j % nobuf
            if j >= nobuf:
                wrt[j - nobuf].wait()
            gath[j][0].wait()
            gath[j][1].wait()

            def _row(r, _, s=s, o=o):
                for g in range(HALF // _L):
                    sl = pl.ds(g * _L, _L)
                    w1 = b1[s, r, sl]
                    w2 = b2[s, r, sl]
                    lo = (plsc.bitcast(jnp.left_shift(w1, 16), jnp.float32)
                          + plsc.bitcast(jnp.left_shift(w2, 16), jnp.float32))
                    hi = (plsc.bitcast(jnp.bitwise_and(w1, _HIMASK),
                                       jnp.float32)
                          + plsc.bitcast(jnp.bitwise_and(w2, _HIMASK),
                                         jnp.float32))
                    ob[o, r, pl.ds(g * _L, _L)] = lo
                    ob[o, r, pl.ds(HALF + g * _L, _L)] = hi
                return 0

            lax.fori_loop(0, chunk, _row, 0)
            off = j * chunk
            wrt[j] = pltpu.async_copy(
                ob.at[o], g_hbm.at[pl.ds(base + off, chunk)], wsem.at[o])
        for j in range(max(nchunk - nobuf, 0), nchunk):
            wrt[j].wait()

    return _sc_gather


_SC_GATHERS = {ds: _make_sc_gather(ds) for ds in sorted(set(SPLITS))}


def _mm_body(x_ref, wa_ref, g_ref, b_ref, acc_ref, o_ref):
    j = pl.program_id(0)

    @pl.when(j == 0)
    def _init():
        o_ref[...] = acc_ref[...] + jnp.broadcast_to(b_ref[...], o_ref.shape)

    x16 = x_ref[...].astype(jnp.bfloat16)
    wa16 = wa_ref[...].astype(jnp.bfloat16)
    g16 = g_ref[...].astype(jnp.bfloat16)
    acc = lax.dot_general(x16, wa16, (((1,), (1,)), ((), ())),
                          preferred_element_type=jnp.float32)
    acc += jnp.dot(x16, g16, preferred_element_type=jnp.float32)
    o_ref[...] += acc


def _matmul_slab(start, dsplit, x, W_affine, G, bcast, acc):
    grid = (dsplit // BD,)
    off = start // BD
    return pl.pallas_call(
        _mm_body,
        grid=grid,
        in_specs=[
            pl.BlockSpec((N, BD), lambda j: (0, off + j)),
            pl.BlockSpec((BASE, BD), lambda j: (0, off + j)),
            pl.BlockSpec((BD, BASE), lambda j: (j, 0)),
            pl.BlockSpec((1, BASE), lambda j: (0, 0)),
            pl.BlockSpec((N, BASE), lambda j: (0, 0)),
        ],
        out_specs=pl.BlockSpec((N, BASE), lambda j: (0, 0)),
        out_shape=jax.ShapeDtypeStruct((N, BASE), jnp.float32),
        compiler_params=pltpu.CompilerParams(
            dimension_semantics=("arbitrary",),
        ),
    )(x, W_affine, G, bcast, acc)


def _pack_table(W):
    Wb = W.astype(jnp.bfloat16)
    pair = jnp.stack([Wb[:, :HALF], Wb[:, HALF:]], axis=-1)
    return lax.bitcast_convert_type(pair, jnp.int32)


def kernel(x, W_affine, b_affine, W1, W2, f1, f2):
    zeros = jnp.zeros((1, BASE), jnp.float32)
    W1p, W2p = _pack_table(W1), _pack_table(W2)
    starts = [sum(SPLITS[:k]) for k in range(len(SPLITS))]
    gs = []
    for k, ds in enumerate(SPLITS):
        sl = slice(starts[k], starts[k] + ds)
        gs.append(_SC_GATHERS[ds](W1p, W2p, f1[sl], f2[sl]))
    acc = jnp.zeros((N, BASE), jnp.float32)
    for k, ds in enumerate(SPLITS):
        bcast = b_affine.reshape(1, BASE) if k == 0 else zeros
        acc = _matmul_slab(starts[k], ds, x, W_affine, gs[k], bcast, acc)
    return acc


# speedup vs baseline: 1.0906x; 1.0906x over previous
import functools

import jax
import jax.numpy as jnp
from jax import lax
from jax.experimental import pallas as pl
from jax.experimental.pallas import tpu as pltpu
from jax.experimental.pallas import tpu_sc as plsc

D = 49152
N = 1024
BASE = 128
HALF = BASE // 2
BD = 1536
SPLITS = (12288, 12288, 12288, 12288)

_INFO = plsc.get_sparse_core_info()
_NC, _NS, _L = _INFO.num_cores, _INFO.num_subcores, _INFO.num_lanes
_NW = _NC * _NS
_HIMASK = -65536


_sc_mesh = plsc.VectorSubcoreMesh(core_axis_name="c", subcore_axis_name="s")


def _make_sc_gather(dsplit):
    rpw = dsplit // _NW
    chunk = 128 if rpw % 128 == 0 else 64
    nchunk = rpw // chunk
    nbuf = min(nchunk, 3)
    nobuf = min(nchunk, 2)

    @functools.partial(
        pl.kernel,
        mesh=_sc_mesh,
        out_type=jax.ShapeDtypeStruct((dsplit, BASE), jnp.float32),
        scratch_types=[
            pltpu.VMEM((rpw,), jnp.int32),
            pltpu.VMEM((rpw,), jnp.int32),
            pltpu.VMEM((nbuf, chunk, HALF), jnp.int32),
            pltpu.VMEM((nbuf, chunk, HALF), jnp.int32),
            pltpu.VMEM((nobuf, chunk, BASE), jnp.float32),
            pltpu.SemaphoreType.DMA((nbuf,)),
            pltpu.SemaphoreType.DMA((nobuf,)),
        ],
        compiler_params=pltpu.CompilerParams(use_tc_tiling_on_sc=False,
                                             needs_layout_passes=False),
    )
    def _sc_gather(w1_hbm, w2_hbm, f1_hbm, f2_hbm, g_hbm,
                   idx1_v, idx2_v, b1, b2, ob, gsem, wsem):
        wid = lax.axis_index("s") * _NC + lax.axis_index("c")
        base = wid * rpw
        pltpu.sync_copy(f1_hbm.at[pl.ds(base, rpw)], idx1_v)
        pltpu.sync_copy(f2_hbm.at[pl.ds(base, rpw)], idx2_v)

        gath = []
        for j in range(nchunk):
            s = j % nbuf
            off = j * chunk
            gath.append((
                pltpu.async_copy(
                    w1_hbm.at[idx1_v.at[pl.ds(off, chunk)]], b1.at[s],
                    gsem.at[s]),
                pltpu.async_copy(
                    w2_hbm.at[idx2_v.at[pl.ds(off, chunk)]], b2.at[s],
                    gsem.at[s]),
            ))

        wrt = [None] * nchunk
        for j in range(nchunk):
            s = j % nbuf
            o = j % nobuf
            if j >= nobuf:
                wrt[j - nobuf].wait()
            gath[j][0].wait()
            gath[j][1].wait()

            def _row(r, _, s=s, o=o):
                for g in range(HALF // _L):
                    sl = pl.ds(g * _L, _L)
                    w1 = b1[s, r, sl]
                    w2 = b2[s, r, sl]
                    lo = (plsc.bitcast(jnp.left_shift(w1, 16), jnp.float32)
                          + plsc.bitcast(jnp.left_shift(w2, 16), jnp.float32))
                    hi = (plsc.bitcast(jnp.bitwise_and(w1, _HIMASK),
                                       jnp.float32)
                          + plsc.bitcast(jnp.bitwise_and(w2, _HIMASK),
                                         jnp.float32))
                    ob[o, r, pl.ds(g * _L, _L)] = lo
                    ob[o, r, pl.ds(HALF + g * _L, _L)] = hi
                return 0

            lax.fori_loop(0, chunk, _row, 0)
            off = j * chunk
            wrt[j] = pltpu.async_copy(
                ob.at[o], g_hbm.at[pl.ds(base + off, chunk)], wsem.at[o])
        for j in range(max(nchunk - nobuf, 0), nchunk):
            wrt[j].wait()

    return _sc_gather


_SC_GATHERS = {ds: _make_sc_gather(ds) for ds in sorted(set(SPLITS))}


def _mm_body(x_ref, wa_ref, g_ref, b_ref, acc_ref, o_ref):
    j = pl.program_id(0)

    @pl.when(j == 0)
    def _init():
        o_ref[...] = acc_ref[...] + jnp.broadcast_to(b_ref[...], o_ref.shape)

    x16 = x_ref[...].astype(jnp.bfloat16)
    wa16 = wa_ref[...].astype(jnp.bfloat16)
    g16 = g_ref[...].astype(jnp.bfloat16)
    acc = lax.dot_general(x16, wa16, (((1,), (1,)), ((), ())),
                          preferred_element_type=jnp.float32)
    acc += jnp.dot(x16, g16, preferred_element_type=jnp.float32)
    o_ref[...] += acc


def _matmul_slab(start, dsplit, x, W_affine, G, bcast, acc):
    grid = (dsplit // BD,)
    off = start // BD
    return pl.pallas_call(
        _mm_body,
        grid=grid,
        in_specs=[
            pl.BlockSpec((N, BD), lambda j: (0, off + j)),
            pl.BlockSpec((BASE, BD), lambda j: (0, off + j)),
            pl.BlockSpec((BD, BASE), lambda j: (j, 0)),
            pl.BlockSpec((1, BASE), lambda j: (0, 0)),
            pl.BlockSpec((N, BASE), lambda j: (0, 0)),
        ],
        out_specs=pl.BlockSpec((N, BASE), lambda j: (0, 0)),
        out_shape=jax.ShapeDtypeStruct((N, BASE), jnp.float32),
        compiler_params=pltpu.CompilerParams(
            dimension_semantics=("arbitrary",),
        ),
    )(x, W_affine, G, bcast, acc)


def _pack_table(W):
    Wb = W.astype(jnp.bfloat16)
    pair = jnp.stack([Wb[:, :HALF], Wb[:, HALF:]], axis=-1)
    return lax.bitcast_convert_type(pair, jnp.int32)


def kernel(x, W_affine, b_affine, W1, W2, f1, f2):
    zeros = jnp.zeros((1, BASE), jnp.float32)
    W1p, W2p = _pack_table(W1), _pack_table(W2)
    starts = [sum(SPLITS[:k]) for k in range(len(SPLITS))]
    gs = []
    for k, ds in enumerate(SPLITS):
        sl = slice(starts[k], starts[k] + ds)
        gs.append(_SC_GATHERS[ds](W1p, W2p, f1[sl], f2[sl]))
    acc = jnp.zeros((N, BASE), jnp.float32)
    for k, ds in enumerate(SPLITS):
        bcast = b_affine.reshape(1, BASE) if k == 0 else zeros
        acc = _matmul_slab(starts[k], ds, x, W_affine, gs[k], bcast, acc)
    return acc


# speedup vs baseline: 1.1242x; 1.0309x over previous
import functools

import jax
import jax.numpy as jnp
from jax import lax
from jax.experimental import pallas as pl
from jax.experimental.pallas import tpu as pltpu
from jax.experimental.pallas import tpu_sc as plsc

D = 49152
N = 1024
BASE = 128
HALF = BASE // 2
BD = 3072
SPLITS = (12288, 12288, 12288, 12288)

_INFO = plsc.get_sparse_core_info()
_NC, _NS, _L = _INFO.num_cores, _INFO.num_subcores, _INFO.num_lanes
_NW = _NC * _NS
_HIMASK = -65536


_sc_mesh = plsc.VectorSubcoreMesh(core_axis_name="c", subcore_axis_name="s")


def _make_sc_gather(dsplit):
    rpw = dsplit // _NW
    chunk = 128 if rpw % 128 == 0 else 64
    nchunk = rpw // chunk
    nbuf = min(nchunk, 3)
    nobuf = min(nchunk, 2)

    @functools.partial(
        pl.kernel,
        mesh=_sc_mesh,
        out_type=jax.ShapeDtypeStruct((dsplit, BASE), jnp.float32),
        scratch_types=[
            pltpu.VMEM((rpw,), jnp.int32),
            pltpu.VMEM((rpw,), jnp.int32),
            pltpu.VMEM((nbuf, chunk, HALF), jnp.int32),
            pltpu.VMEM((nbuf, chunk, HALF), jnp.int32),
            pltpu.VMEM((nobuf, chunk, BASE), jnp.float32),
            pltpu.SemaphoreType.DMA((nbuf,)),
            pltpu.SemaphoreType.DMA((nobuf,)),
        ],
        compiler_params=pltpu.CompilerParams(use_tc_tiling_on_sc=False,
                                             needs_layout_passes=False),
    )
    def _sc_gather(w1_hbm, w2_hbm, f1_hbm, f2_hbm, g_hbm,
                   idx1_v, idx2_v, b1, b2, ob, gsem, wsem):
        wid = lax.axis_index("s") * _NC + lax.axis_index("c")
        base = wid * rpw
        pltpu.sync_copy(f1_hbm.at[pl.ds(base, rpw)], idx1_v)
        pltpu.sync_copy(f2_hbm.at[pl.ds(base, rpw)], idx2_v)

        gath = []
        for j in range(nchunk):
            s = j % nbuf
            off = j * chunk
            gath.append((
                pltpu.async_copy(
                    w1_hbm.at[idx1_v.at[pl.ds(off, chunk)]], b1.at[s],
                    gsem.at[s]),
                pltpu.async_copy(
                    w2_hbm.at[idx2_v.at[pl.ds(off, chunk)]], b2.at[s],
                    gsem.at[s]),
            ))

        wrt = [None] * nchunk
        for j in range(nchunk):
            s = j % nbuf
            o = j % nobuf
            if j >= nobuf:
                wrt[j - nobuf].wait()
            gath[j][0].wait()
            gath[j][1].wait()

            def _row(r, _, s=s, o=o):
                for g in range(HALF // _L):
                    sl = pl.ds(g * _L, _L)
                    w1 = b1[s, r, sl]
                    w2 = b2[s, r, sl]
                    lo = (plsc.bitcast(jnp.left_shift(w1, 16), jnp.float32)
                          + plsc.bitcast(jnp.left_shift(w2, 16), jnp.float32))
                    hi = (plsc.bitcast(jnp.bitwise_and(w1, _HIMASK),
                                       jnp.float32)
                          + plsc.bitcast(jnp.bitwise_and(w2, _HIMASK),
                                         jnp.float32))
                    ob[o, r, pl.ds(g * _L, _L)] = lo
                    ob[o, r, pl.ds(HALF + g * _L, _L)] = hi
                return 0

            lax.fori_loop(0, chunk, _row, 0)
            off = j * chunk
            wrt[j] = pltpu.async_copy(
                ob.at[o], g_hbm.at[pl.ds(base + off, chunk)], wsem.at[o])
        for j in range(max(nchunk - nobuf, 0), nchunk):
            wrt[j].wait()

    return _sc_gather


_SC_GATHERS = {ds: _make_sc_gather(ds) for ds in sorted(set(SPLITS))}


def _mm_body(x_ref, wa_ref, g_ref, b_ref, acc_ref, o_ref):
    j = pl.program_id(0)

    @pl.when(j == 0)
    def _init():
        o_ref[...] = acc_ref[...] + jnp.broadcast_to(b_ref[...], o_ref.shape)

    x16 = x_ref[...].astype(jnp.bfloat16)
    wa16 = wa_ref[...].astype(jnp.bfloat16)
    g16 = g_ref[...].astype(jnp.bfloat16)
    acc = lax.dot_general(x16, wa16, (((1,), (1,)), ((), ())),
                          preferred_element_type=jnp.float32)
    acc += jnp.dot(x16, g16, preferred_element_type=jnp.float32)
    o_ref[...] += acc


def _matmul_slab(start, dsplit, x, W_affine, G, bcast, acc):
    grid = (dsplit // BD,)
    off = start // BD
    return pl.pallas_call(
        _mm_body,
        grid=grid,
        in_specs=[
            pl.BlockSpec((N, BD), lambda j: (0, off + j)),
            pl.BlockSpec((BASE, BD), lambda j: (0, off + j)),
            pl.BlockSpec((BD, BASE), lambda j: (j, 0)),
            pl.BlockSpec((1, BASE), lambda j: (0, 0)),
            pl.BlockSpec((N, BASE), lambda j: (0, 0)),
        ],
        out_specs=pl.BlockSpec((N, BASE), lambda j: (0, 0)),
        out_shape=jax.ShapeDtypeStruct((N, BASE), jnp.float32),
        compiler_params=pltpu.CompilerParams(
            dimension_semantics=("arbitrary",),
        ),
    )(x, W_affine, G, bcast, acc)


def _pack_table(W):
    Wb = W.astype(jnp.bfloat16)
    pair = jnp.stack([Wb[:, :HALF], Wb[:, HALF:]], axis=-1)
    return lax.bitcast_convert_type(pair, jnp.int32)


def kernel(x, W_affine, b_affine, W1, W2, f1, f2):
    zeros = jnp.zeros((1, BASE), jnp.float32)
    W1p, W2p = _pack_table(W1), _pack_table(W2)
    starts = [sum(SPLITS[:k]) for k in range(len(SPLITS))]
    gs = []
    for k, ds in enumerate(SPLITS):
        sl = slice(starts[k], starts[k] + ds)
        gs.append(_SC_GATHERS[ds](W1p, W2p, f1[sl], f2[sl]))
    acc = jnp.zeros((N, BASE), jnp.float32)
    for k, ds in enumerate(SPLITS):
        bcast = b_affine.reshape(1, BASE) if k == 0 else zeros
        acc = _matmul_slab(starts[k], ds, x, W_affine, gs[k], bcast, acc)
    return acc
